# baseline (speedup 1.0000x reference)
import dataclasses
import functools

import jax
import jax.numpy as jnp
from jax import lax
from jax.experimental import pallas as pl
from jax.experimental.pallas import tpu as pltpu
from jax.experimental.pallas import tpu_sc as plsc

N = 10000
E = 320000
R = 16
D = 128

N_PAD = 10240
BLK = 1024
NWORK = 32
EC = E // NWORK
NC = N_PAD // NWORK
GW = 64

_mesh = plsc.VectorSubcoreMesh(core_axis_name="c", subcore_axis_name="s")

_cp = pltpu.CompilerParams()
if "needs_layout_passes" in pltpu.CompilerParams.__dataclass_fields__:
    _cp = dataclasses.replace(_cp, needs_layout_passes=False)


NSUB = 8


@functools.partial(
    pl.kernel,
    out_type=jax.ShapeDtypeStruct((NWORK * N_PAD,), jnp.int32),
    mesh=_mesh,
    scratch_types=[
        pltpu.VMEM((EC,), jnp.int32),
        pltpu.VMEM((NSUB * N_PAD,), jnp.int32),
        pltpu.VMEM((N_PAD,), jnp.int32),
        pltpu.SemaphoreType.DMA,
    ],
    compiler_params=_cp,
)
def _sc_lastedge(dst_hbm, neg_hbm, tbl_hbm, dst_v, tbl8_v, tbl_v, sem):
    wid = lax.axis_index("s") * 2 + lax.axis_index("c")
    base = wid * EC
    cp_dst = pltpu.async_copy(dst_hbm.at[pl.ds(base, EC)], dst_v, sem)
    cp_init = pltpu.async_copy(neg_hbm, tbl8_v, sem)
    cp_dst.wait()
    cp_init.wait()

    lane = lax.iota(jnp.int32, 16)
    off8 = (lane & 7) * N_PAD
    lo = lane < 8
    hi = lane >= 8

    @pl.loop(0, EC // 16)
    def _(v):
        d = dst_v[pl.ds(v * 16, 16)]
        idx = off8 + d
        val = (base + v * 16) + lane
        plsc.store_scatter(tbl8_v, [idx], val, mask=lo)
        plsc.store_scatter(tbl8_v, [idx], val, mask=hi)

    @pl.loop(0, N_PAD // 16)
    def _(k):
        m = tbl8_v[pl.ds(k * 16, 16)]
        for c in range(1, NSUB):
            m = jnp.maximum(m, tbl8_v[pl.ds(c * N_PAD + k * 16, 16)])
        tbl_v[pl.ds(k * 16, 16)] = m

    pltpu.sync_copy(tbl_v, tbl_hbm.at[pl.ds(wid * N_PAD, N_PAD)])


@functools.partial(
    pl.kernel,
    out_type=(
        jax.ShapeDtypeStruct((N_PAD, D), jnp.float32),
        jax.ShapeDtypeStruct((N_PAD,), jnp.int32),
    ),
    mesh=_mesh,
    scratch_types=[
        pltpu.VMEM((NWORK * NC,), jnp.int32),
        pltpu.VMEM((NC,), jnp.int32),
        pltpu.VMEM((NC,), jnp.int32),
        pltpu.VMEM((NC,), jnp.int32),
        pltpu.VMEM((NC,), jnp.int32),
        pltpu.VMEM((NC,), jnp.int32),
        pltpu.VMEM((NC,), jnp.int32),
        pltpu.VMEM((NC, D), jnp.float32),
        pltpu.SemaphoreType.DMA,
    ],
    compiler_params=_cp,
)
def _sc_gather(tbl_hbm, src_hbm, rel_hbm, h_hbm, x_hbm, relout_hbm,
               tb_v, eidx_v, m_v, sg_v, rg_v, sidx_v, rout_v, rows_v, sem):
    wid = lax.axis_index("s") * 2 + lax.axis_index("c")
    nbase = wid * NC
    cps = []
    for j in range(NWORK):
        cps.append(pltpu.async_copy(
            tbl_hbm.at[pl.ds(j * N_PAD + nbase, NC)],
            tb_v.at[pl.ds(j * NC, NC)], sem))
    for cp in cps:
        cp.wait()

    @pl.loop(0, NC // 16)
    def _(k):
        m = tb_v[pl.ds(k * 16, 16)]
        for j in range(1, NWORK):
            m = jnp.maximum(m, tb_v[pl.ds(j * NC + k * 16, 16)])
        sl = pl.ds(k * 16, 16)
        m_v[sl] = m
        eidx_v[sl] = jnp.maximum(m, 0)

    cps = []
    for c in range(NC // GW):
        sl = pl.ds(c * GW, GW)
        cps.append(pltpu.async_copy(src_hbm.at[eidx_v.at[sl]], sg_v.at[sl], sem))
        cps.append(pltpu.async_copy(rel_hbm.at[eidx_v.at[sl]], rg_v.at[sl], sem))
    for cp in cps:
        cp.wait()

    @pl.loop(0, NC // 16)
    def _(k):
        sl = pl.ds(k * 16, 16)
        valid = m_v[sl] >= 0
        sidx_v[sl] = jnp.where(valid, sg_v[sl], 0)
        rout_v[sl] = jnp.where(valid, rg_v[sl], R)

    cps = []
    for c in range(NC // GW):
        sl = pl.ds(c * GW, GW)
        cps.append(pltpu.async_copy(h_hbm.at[sidx_v.at[sl]], rows_v.at[sl], sem))
    for cp in cps:
        cp.wait()

    pltpu.sync_copy(rows_v, x_hbm.at[pl.ds(nbase, NC)])
    pltpu.sync_copy(rout_v, relout_hbm.at[pl.ds(nbase, NC)])


def _tc_body(h_ref, x_ref, rel_ref, w_ref, slw_ref, o_ref):
    acc = jnp.dot(h_ref[...], slw_ref[...], preferred_element_type=jnp.float32)
    relv = rel_ref[...]
    x = x_ref[...]
    for r in range(R):
        m = (relv == r).astype(jnp.float32)
        acc = acc + jnp.dot(x * m, w_ref[r], preferred_element_type=jnp.float32)
    o_ref[...] = acc


def _tc_matmul(h, x_pad, rel2, wb, slwb):
    nb = N_PAD // BLK
    return pl.pallas_call(
        _tc_body,
        grid=(nb,),
        in_specs=[
            pl.BlockSpec((BLK, D), lambda i: (i, 0)),
            pl.BlockSpec((BLK, D), lambda i: (i, 0)),
            pl.BlockSpec((BLK, 1), lambda i: (i, 0)),
            pl.BlockSpec((R, D, D), lambda i: (0, 0, 0)),
            pl.BlockSpec((D, D), lambda i: (0, 0)),
        ],
        out_specs=pl.BlockSpec((BLK, D), lambda i: (i, 0)),
        out_shape=jax.ShapeDtypeStruct((N, D), jnp.float32),
    )(h, x_pad, rel2, wb, slwb)


def kernel(h, edges, weight, self_loop_weight):
    src = edges[:, 0]
    rel = edges[:, 1]
    dst = edges[:, 2]

    neg = jnp.full((NSUB * N_PAD,), -1, jnp.int32)
    tbl = _sc_lastedge(dst, neg)
    x, relv = _sc_gather(tbl, src, rel, h)
    rel2 = relv.reshape(N_PAD, 1)

    return _tc_matmul(h, x, rel2, weight, self_loop_weight)


# speedup vs baseline: 1.0704x; 1.0704x over previous
import dataclasses
import functools

import jax
import jax.numpy as jnp
from jax import lax
from jax.experimental import pallas as pl
from jax.experimental.pallas import tpu as pltpu
from jax.experimental.pallas import tpu_sc as plsc

N = 10000
E = 320000
R = 16
D = 128

N_PAD = 10240
BLK = 1024
NWORK = 32
EC = E // NWORK
NC = N_PAD // NWORK
GW = 64

_mesh = plsc.VectorSubcoreMesh(core_axis_name="c", subcore_axis_name="s")

_cp = pltpu.CompilerParams()
if "needs_layout_passes" in pltpu.CompilerParams.__dataclass_fields__:
    _cp = dataclasses.replace(_cp, needs_layout_passes=False)


@functools.partial(
    pl.kernel,
    out_type=jax.ShapeDtypeStruct((NWORK * N_PAD,), jnp.int32),
    mesh=_mesh,
    scratch_types=[
        pltpu.VMEM((EC,), jnp.int32),
        pltpu.VMEM((N_PAD,), jnp.int32),
    ],
    compiler_params=_cp,
)
def _sc_lastedge(dst_hbm, tbl_hbm, dst_v, tbl_v):
    wid = lax.axis_index("s") * 2 + lax.axis_index("c")
    base = wid * EC
    pltpu.sync_copy(dst_hbm.at[pl.ds(base, EC)], dst_v)

    neg1 = jnp.full((16,), -1, jnp.int32)

    @pl.loop(0, N_PAD // 16)
    def _(i):
        tbl_v[pl.ds(i * 16, 16)] = neg1

    lane = lax.iota(jnp.int32, 16)
    shift_idx = jnp.minimum(lane + 1, 15)
    last_lane = lane == 15

    @pl.loop(0, EC // 80)
    def _(v):
        for u in range(5):
            d = dst_v[pl.ds(v * 80 + u * 16, 16)]
            key = (d << 4) + lane
            val = (base + v * 80 + u * 16) + lane
            ks, vs = plsc.sort_key_val(key, val)
            ds_ = ks >> 4
            nxt = ds_.at[shift_idx].get(mode="promise_in_bounds")
            winner = (ds_ != nxt) | last_lane
            plsc.store_scatter(tbl_v, [ds_], vs, mask=winner)

    pltpu.sync_copy(tbl_v, tbl_hbm.at[pl.ds(wid * N_PAD, N_PAD)])


@functools.partial(
    pl.kernel,
    out_type=(
        jax.ShapeDtypeStruct((N_PAD, D), jnp.float32),
        jax.ShapeDtypeStruct((N_PAD,), jnp.int32),
    ),
    mesh=_mesh,
    scratch_types=[
        pltpu.VMEM((NWORK * NC,), jnp.int32),
        pltpu.VMEM((NC,), jnp.int32),
        pltpu.VMEM((NC,), jnp.int32),
        pltpu.VMEM((NC,), jnp.int32),
        pltpu.VMEM((NC,), jnp.int32),
        pltpu.VMEM((NC,), jnp.int32),
        pltpu.VMEM((NC,), jnp.int32),
        pltpu.VMEM((NC, D), jnp.float32),
        pltpu.SemaphoreType.DMA,
    ],
    compiler_params=_cp,
)
def _sc_gather(tbl_hbm, src_hbm, rel_hbm, h_hbm, x_hbm, relout_hbm,
               tb_v, eidx_v, m_v, sg_v, rg_v, sidx_v, rout_v, rows_v, sem):
    wid = lax.axis_index("s") * 2 + lax.axis_index("c")
    nbase = wid * NC
    cps = []
    for j in range(NWORK):
        cps.append(pltpu.async_copy(
            tbl_hbm.at[pl.ds(j * N_PAD + nbase, NC)],
            tb_v.at[pl.ds(j * NC, NC)], sem))
    for cp in cps:
        cp.wait()

    @pl.loop(0, NC // 16)
    def _(k):
        m = tb_v[pl.ds(k * 16, 16)]
        for j in range(1, NWORK):
            m = jnp.maximum(m, tb_v[pl.ds(j * NC + k * 16, 16)])
        sl = pl.ds(k * 16, 16)
        m_v[sl] = m
        eidx_v[sl] = jnp.maximum(m, 0)

    cps = []
    for c in range(NC // GW):
        sl = pl.ds(c * GW, GW)
        cps.append(pltpu.async_copy(src_hbm.at[eidx_v.at[sl]], sg_v.at[sl], sem))
        cps.append(pltpu.async_copy(rel_hbm.at[eidx_v.at[sl]], rg_v.at[sl], sem))
    for cp in cps:
        cp.wait()

    @pl.loop(0, NC // 16)
    def _(k):
        sl = pl.ds(k * 16, 16)
        valid = m_v[sl] >= 0
        sidx_v[sl] = jnp.where(valid, sg_v[sl], 0)
        rout_v[sl] = jnp.where(valid, rg_v[sl], R)

    cps = []
    for c in range(NC // GW):
        sl = pl.ds(c * GW, GW)
        cps.append(pltpu.async_copy(h_hbm.at[sidx_v.at[sl]], rows_v.at[sl], sem))
    for cp in cps:
        cp.wait()

    pltpu.sync_copy(rows_v, x_hbm.at[pl.ds(nbase, NC)])
    pltpu.sync_copy(rout_v, relout_hbm.at[pl.ds(nbase, NC)])


def _tc_self_body(h_ref, slw_ref, o_ref):
    o_ref[...] = jnp.dot(h_ref[...], slw_ref[...],
                         preferred_element_type=jnp.float32)


def _tc_self(h, slw):
    return pl.pallas_call(
        _tc_self_body,
        grid=(N_PAD // BLK,),
        in_specs=[
            pl.BlockSpec((BLK, D), lambda i: (i, 0)),
            pl.BlockSpec((D, D), lambda i: (0, 0)),
        ],
        out_specs=pl.BlockSpec((BLK, D), lambda i: (i, 0)),
        out_shape=jax.ShapeDtypeStruct((N, D), jnp.float32),
    )(h, slw)


def _tc_body(self_ref, x_ref, rel_ref, w_ref, o_ref):
    acc = self_ref[...]
    relv = rel_ref[...]
    x = x_ref[...]
    for r in range(R):
        m = (relv == r).astype(jnp.float32)
        acc = acc + jnp.dot(x * m, w_ref[r], preferred_element_type=jnp.float32)
    o_ref[...] = acc


def _tc_matmul(selfout, x_pad, rel2, w):
    nb = N_PAD // BLK
    return pl.pallas_call(
        _tc_body,
        grid=(nb,),
        in_specs=[
            pl.BlockSpec((BLK, D), lambda i: (i, 0)),
            pl.BlockSpec((BLK, D), lambda i: (i, 0)),
            pl.BlockSpec((BLK, 1), lambda i: (i, 0)),
            pl.BlockSpec((R, D, D), lambda i: (0, 0, 0)),
        ],
        out_specs=pl.BlockSpec((BLK, D), lambda i: (i, 0)),
        out_shape=jax.ShapeDtypeStruct((N, D), jnp.float32),
    )(selfout, x_pad, rel2, w)


def kernel(h, edges, weight, self_loop_weight):
    src = edges[:, 0]
    rel = edges[:, 1]
    dst = edges[:, 2]

    tbl = _sc_lastedge(dst)
    x, relv = _sc_gather(tbl, src, rel, h)
    rel2 = relv.reshape(N_PAD, 1)

    selfout = _tc_self(h, self_loop_weight)
    return _tc_matmul(selfout, x, rel2, weight)


# speedup vs baseline: 1.0900x; 1.0183x over previous
import dataclasses
import functools

import jax
import jax.numpy as jnp
from jax import lax
from jax.experimental import pallas as pl
from jax.experimental.pallas import tpu as pltpu
from jax.experimental.pallas import tpu_sc as plsc

N = 10000
E = 320000
R = 16
D = 128

N_PAD = 10240
BLK = 1024
NWORK = 32
EC = E // NWORK
NC = N_PAD // NWORK
GW = 64

_mesh = plsc.VectorSubcoreMesh(core_axis_name="c", subcore_axis_name="s")

_cp = pltpu.CompilerParams()
if "needs_layout_passes" in pltpu.CompilerParams.__dataclass_fields__:
    _cp = dataclasses.replace(_cp, needs_layout_passes=False)


@functools.partial(
    pl.kernel,
    out_type=jax.ShapeDtypeStruct((NWORK * N_PAD,), jnp.int32),
    mesh=_mesh,
    scratch_types=[
        pltpu.VMEM((EC,), jnp.int32),
        pltpu.VMEM((N_PAD,), jnp.int32),
    ],
    compiler_params=_cp,
)
def _sc_lastedge(dst_hbm, tbl_hbm, dst_v, tbl_v):
    wid = lax.axis_index("s") * 2 + lax.axis_index("c")
    base = wid * EC
    pltpu.sync_copy(dst_hbm.at[pl.ds(base, EC)], dst_v)

    neg1 = jnp.full((16,), -1, jnp.int32)

    @pl.loop(0, N_PAD // 16)
    def _(i):
        tbl_v[pl.ds(i * 16, 16)] = neg1

    lane = lax.iota(jnp.int32, 16)
    shift_idx = jnp.minimum(lane + 1, 15)
    last_lane = lane == 15

    @pl.loop(0, EC // 80)
    def _(v):
        for u in range(5):
            d = dst_v[pl.ds(v * 80 + u * 16, 16)]
            key = (d << 4) + lane
            val = (base + v * 80 + u * 16) + lane
            ks, vs = plsc.sort_key_val(key, val)
            ds_ = ks >> 4
            nxt = ds_.at[shift_idx].get(mode="promise_in_bounds")
            winner = (ds_ != nxt) | last_lane
            plsc.store_scatter(tbl_v, [ds_], vs, mask=winner)

    pltpu.sync_copy(tbl_v, tbl_hbm.at[pl.ds(wid * N_PAD, N_PAD)])


NCH = NC // GW


@functools.partial(
    pl.kernel,
    out_type=(
        jax.ShapeDtypeStruct((N_PAD, D), jnp.float32),
        jax.ShapeDtypeStruct((N_PAD,), jnp.int32),
    ),
    mesh=_mesh,
    scratch_types=(
        [
            pltpu.VMEM((NWORK * NC,), jnp.int32),
            pltpu.VMEM((NC,), jnp.int32),
            pltpu.VMEM((NC,), jnp.int32),
            pltpu.VMEM((NC,), jnp.int32),
            pltpu.VMEM((NC,), jnp.int32),
            pltpu.VMEM((NC,), jnp.int32),
            pltpu.VMEM((NC,), jnp.int32),
            pltpu.VMEM((NC, D), jnp.float32),
            pltpu.SemaphoreType.DMA,
        ]
        + [pltpu.SemaphoreType.DMA] * NCH
        + [pltpu.SemaphoreType.DMA] * NCH
    ),
    compiler_params=_cp,
)
def _sc_gather(tbl_hbm, src_hbm, rel_hbm, h_hbm, x_hbm, relout_hbm,
               tb_v, eidx_v, m_v, sg_v, rg_v, sidx_v, rout_v, rows_v,
               sem_t, *sems):
    sems_sr = sems[:NCH]
    sems_h = sems[NCH:]
    wid = lax.axis_index("s") * 2 + lax.axis_index("c")
    nbase = wid * NC
    cps = []
    for j in range(NWORK):
        cps.append(pltpu.async_copy(
            tbl_hbm.at[pl.ds(j * N_PAD + nbase, NC)],
            tb_v.at[pl.ds(j * NC, NC)], sem_t))
    for cp in cps:
        cp.wait()

    cps_sr = []
    for c in range(NCH):
        for t in range(GW // 16):
            k = c * (GW // 16) + t
            m = tb_v[pl.ds(k * 16, 16)]
            for j in range(1, NWORK):
                m = jnp.maximum(m, tb_v[pl.ds(j * NC + k * 16, 16)])
            sl = pl.ds(k * 16, 16)
            m_v[sl] = m
            eidx_v[sl] = jnp.maximum(m, 0)
        slc = pl.ds(c * GW, GW)
        cps_sr.append((
            pltpu.async_copy(src_hbm.at[eidx_v.at[slc]], sg_v.at[slc],
                             sems_sr[c]),
            pltpu.async_copy(rel_hbm.at[eidx_v.at[slc]], rg_v.at[slc],
                             sems_sr[c]),
        ))

    cps_h = []
    for c in range(NCH):
        cps_sr[c][0].wait()
        cps_sr[c][1].wait()
        for t in range(GW // 16):
            k = c * (GW // 16) + t
            sl = pl.ds(k * 16, 16)
            valid = m_v[sl] >= 0
            sidx_v[sl] = jnp.where(valid, sg_v[sl], 0)
            rout_v[sl] = jnp.where(valid, rg_v[sl], R)
        slc = pl.ds(c * GW, GW)
        cps_h.append(pltpu.async_copy(
            h_hbm.at[sidx_v.at[slc]], rows_v.at[slc], sems_h[c]))

    for c in range(NCH):
        cps_h[c].wait()
        pltpu.sync_copy(rows_v.at[pl.ds(c * GW, GW)],
                        x_hbm.at[pl.ds(nbase + c * GW, GW)])

    pltpu.sync_copy(rout_v, relout_hbm.at[pl.ds(nbase, NC)])


def _tc_body(h_ref, x_ref, rel_ref, w_ref, slw_ref, o_ref):
    acc = jnp.dot(h_ref[...], slw_ref[...], preferred_element_type=jnp.float32)
    relv = rel_ref[...]
    x = x_ref[...]
    for r in range(R):
        m = (relv == r).astype(jnp.float32)
        acc = acc + jnp.dot(x * m, w_ref[r], preferred_element_type=jnp.float32)
    o_ref[...] = acc


def _tc_matmul(h, x_pad, rel2, w, slw):
    nb = N_PAD // BLK
    return pl.pallas_call(
        _tc_body,
        grid=(nb,),
        in_specs=[
            pl.BlockSpec((BLK, D), lambda i: (i, 0)),
            pl.BlockSpec((BLK, D), lambda i: (i, 0)),
            pl.BlockSpec((BLK, 1), lambda i: (i, 0)),
            pl.BlockSpec((R, D, D), lambda i: (0, 0, 0)),
            pl.BlockSpec((D, D), lambda i: (0, 0)),
        ],
        out_specs=pl.BlockSpec((BLK, D), lambda i: (i, 0)),
        out_shape=jax.ShapeDtypeStruct((N, D), jnp.float32),
    )(h, x_pad, rel2, w, slw)


def kernel(h, edges, weight, self_loop_weight):
    src = edges[:, 0]
    rel = edges[:, 1]
    dst = edges[:, 2]

    tbl = _sc_lastedge(dst)
    x, relv = _sc_gather(tbl, src, rel, h)
    rel2 = relv.reshape(N_PAD, 1)

    return _tc_matmul(h, x, rel2, weight, self_loop_weight)


# speedup vs baseline: 1.1031x; 1.0121x over previous
import dataclasses
import functools

import jax
import jax.numpy as jnp
from jax import lax
from jax.experimental import pallas as pl
from jax.experimental.pallas import tpu as pltpu
from jax.experimental.pallas import tpu_sc as plsc

N = 10000
E = 320000
R = 16
D = 128

N_PAD = 10240
BLK = 1024
NWORK = 32
EC = E // NWORK
NC = N_PAD // NWORK
GW = 64

_mesh = plsc.VectorSubcoreMesh(core_axis_name="c", subcore_axis_name="s")

_cp = pltpu.CompilerParams()
if "needs_layout_passes" in pltpu.CompilerParams.__dataclass_fields__:
    _cp = dataclasses.replace(_cp, needs_layout_passes=False)


@functools.partial(
    pl.kernel,
    out_type=jax.ShapeDtypeStruct((NWORK * N_PAD,), jnp.int32),
    mesh=_mesh,
    scratch_types=[
        pltpu.VMEM((EC,), jnp.int32),
        pltpu.VMEM((N_PAD,), jnp.int32),
        pltpu.SemaphoreType.DMA,
    ],
    compiler_params=_cp,
)
def _sc_lastedge(dst_hbm, tbl_hbm, dst_v, tbl_v, sem):
    wid = lax.axis_index("s") * 2 + lax.axis_index("c")
    base = wid * EC
    cp_dst = pltpu.async_copy(dst_hbm.at[pl.ds(base, EC)], dst_v, sem)

    neg1 = jnp.full((16,), -1, jnp.int32)

    @pl.loop(0, N_PAD // 16)
    def _(i):
        tbl_v[pl.ds(i * 16, 16)] = neg1

    cp_dst.wait()
    lane = lax.iota(jnp.int32, 16)
    shift_idx = jnp.minimum(lane + 1, 15)
    last_lane = lane == 15

    @pl.loop(0, EC // 80)
    def _(v):
        for u in range(5):
            d = dst_v[pl.ds(v * 80 + u * 16, 16)]
            key = (d << 4) + lane
            val = (base + v * 80 + u * 16) + lane
            ks, vs = plsc.sort_key_val(key, val)
            ds_ = ks >> 4
            nxt = ds_.at[shift_idx].get(mode="promise_in_bounds")
            winner = (ds_ != nxt) | last_lane
            plsc.store_scatter(tbl_v, [ds_], vs, mask=winner)

    pltpu.sync_copy(tbl_v, tbl_hbm.at[pl.ds(wid * N_PAD, N_PAD)])


NCH = NC // GW


@functools.partial(
    pl.kernel,
    out_type=(
        jax.ShapeDtypeStruct((N_PAD, D), jnp.float32),
        jax.ShapeDtypeStruct((N_PAD,), jnp.int32),
    ),
    mesh=_mesh,
    scratch_types=(
        [
            pltpu.VMEM((NWORK * NC,), jnp.int32),
            pltpu.VMEM((NC,), jnp.int32),
            pltpu.VMEM((NC,), jnp.int32),
            pltpu.VMEM((NC,), jnp.int32),
            pltpu.VMEM((NC,), jnp.int32),
            pltpu.VMEM((NC,), jnp.int32),
            pltpu.VMEM((NC,), jnp.int32),
            pltpu.VMEM((NC, D), jnp.float32),
            pltpu.SemaphoreType.DMA,
        ]
        + [pltpu.SemaphoreType.DMA] * NCH
        + [pltpu.SemaphoreType.DMA] * NCH
    ),
    compiler_params=_cp,
)
def _sc_gather(tbl_hbm, src_hbm, rel_hbm, h_hbm, x_hbm, relout_hbm,
               tb_v, eidx_v, m_v, sg_v, rg_v, sidx_v, rout_v, rows_v,
               sem_t, *sems):
    sems_sr = sems[:NCH]
    sems_h = sems[NCH:]
    wid = lax.axis_index("s") * 2 + lax.axis_index("c")
    nbase = wid * NC
    cps = []
    for j in range(NWORK):
        cps.append(pltpu.async_copy(
            tbl_hbm.at[pl.ds(j * N_PAD + nbase, NC)],
            tb_v.at[pl.ds(j * NC, NC)], sem_t))
    for cp in cps:
        cp.wait()

    cps_sr = []
    for c in range(NCH):
        for t in range(GW // 16):
            k = c * (GW // 16) + t
            m = tb_v[pl.ds(k * 16, 16)]
            for j in range(1, NWORK):
                m = jnp.maximum(m, tb_v[pl.ds(j * NC + k * 16, 16)])
            sl = pl.ds(k * 16, 16)
            m_v[sl] = m
            eidx_v[sl] = jnp.maximum(m, 0)
        slc = pl.ds(c * GW, GW)
        cps_sr.append((
            pltpu.async_copy(src_hbm.at[eidx_v.at[slc]], sg_v.at[slc],
                             sems_sr[c]),
            pltpu.async_copy(rel_hbm.at[eidx_v.at[slc]], rg_v.at[slc],
                             sems_sr[c]),
        ))

    cps_h = []
    for c in range(NCH):
        cps_sr[c][0].wait()
        cps_sr[c][1].wait()
        for t in range(GW // 16):
            k = c * (GW // 16) + t
            sl = pl.ds(k * 16, 16)
            valid = m_v[sl] >= 0
            sidx_v[sl] = jnp.where(valid, sg_v[sl], 0)
            rout_v[sl] = jnp.where(valid, rg_v[sl], R)
        slc = pl.ds(c * GW, GW)
        cps_h.append(pltpu.async_copy(
            h_hbm.at[sidx_v.at[slc]], rows_v.at[slc], sems_h[c]))

    pltpu.sync_copy(rout_v, relout_hbm.at[pl.ds(nbase, NC)])

    for c in range(NCH):
        cps_h[c].wait()
        pltpu.sync_copy(rows_v.at[pl.ds(c * GW, GW)],
                        x_hbm.at[pl.ds(nbase + c * GW, GW)])


def _tc_body(h_ref, x_ref, rel_ref, w_ref, slw_ref, o_ref):
    acc = jnp.dot(h_ref[...], slw_ref[...], preferred_element_type=jnp.float32)
    relv = rel_ref[...]
    x = x_ref[...]
    for r in range(R):
        m = (relv == r).astype(jnp.float32)
        acc = acc + jnp.dot(x * m, w_ref[r], preferred_element_type=jnp.float32)
    o_ref[...] = acc


def _tc_matmul(h, x_pad, rel2, w, slw):
    nb = N_PAD // BLK
    return pl.pallas_call(
        _tc_body,
        grid=(nb,),
        in_specs=[
            pl.BlockSpec((BLK, D), lambda i: (i, 0)),
            pl.BlockSpec((BLK, D), lambda i: (i, 0)),
            pl.BlockSpec((BLK, 1), lambda i: (i, 0)),
            pl.BlockSpec((R, D, D), lambda i: (0, 0, 0)),
            pl.BlockSpec((D, D), lambda i: (0, 0)),
        ],
        out_specs=pl.BlockSpec((BLK, D), lambda i: (i, 0)),
        out_shape=jax.ShapeDtypeStruct((N, D), jnp.float32),
    )(h, x_pad, rel2, w, slw)


def kernel(h, edges, weight, self_loop_weight):
    src = edges[:, 0]
    rel = edges[:, 1]
    dst = edges[:, 2]

    tbl = _sc_lastedge(dst)
    x, relv = _sc_gather(tbl, src, rel, h)
    rel2 = relv.reshape(N_PAD, 1)

    return _tc_matmul(h, x, rel2, weight, self_loop_weight)


# speedup vs baseline: 1.1040x; 1.0008x over previous
import dataclasses
import functools

import jax
import jax.numpy as jnp
from jax import lax
from jax.experimental import pallas as pl
from jax.experimental.pallas import tpu as pltpu
from jax.experimental.pallas import tpu_sc as plsc

N = 10000
E = 320000
R = 16
D = 128

N_PAD = 10240
BLK = 1024
NWORK = 32
EC = E // NWORK
NC = N_PAD // NWORK
GW = 64

_mesh = plsc.VectorSubcoreMesh(core_axis_name="c", subcore_axis_name="s")

_cp = pltpu.CompilerParams()
if "needs_layout_passes" in pltpu.CompilerParams.__dataclass_fields__:
    _cp = dataclasses.replace(_cp, needs_layout_passes=False)


@functools.partial(
    pl.kernel,
    out_type=jax.ShapeDtypeStruct((NWORK * N_PAD,), jnp.int32),
    mesh=_mesh,
    scratch_types=[
        pltpu.VMEM((EC,), jnp.int32),
        pltpu.VMEM((N_PAD,), jnp.int32),
        pltpu.SemaphoreType.DMA,
    ],
    compiler_params=_cp,
)
def _sc_lastedge(dst_hbm, tbl_hbm, dst_v, tbl_v, sem):
    wid = lax.axis_index("s") * 2 + lax.axis_index("c")
    base = wid * EC
    cp_dst = pltpu.async_copy(dst_hbm.at[pl.ds(base, EC)], dst_v, sem)

    neg1 = jnp.full((16,), -1, jnp.int32)

    @pl.loop(0, N_PAD // 16)
    def _(i):
        tbl_v[pl.ds(i * 16, 16)] = neg1

    cp_dst.wait()
    lane = lax.iota(jnp.int32, 16)
    shift_idx = jnp.minimum(lane + 1, 15)
    last_lane = lane == 15

    @pl.loop(0, EC // 400)
    def _(v):
        for u in range(25):
            d = dst_v[pl.ds(v * 400 + u * 16, 16)]
            key = (d << 4) + lane
            val = (base + v * 400 + u * 16) + lane
            ks, vs = plsc.sort_key_val(key, val)
            ds_ = ks >> 4
            nxt = ds_.at[shift_idx].get(mode="promise_in_bounds")
            winner = (ds_ != nxt) | last_lane
            plsc.store_scatter(tbl_v, [ds_], vs, mask=winner)

    pltpu.sync_copy(tbl_v, tbl_hbm.at[pl.ds(wid * N_PAD, N_PAD)])


NCH = NC // GW


@functools.partial(
    pl.kernel,
    out_type=(
        jax.ShapeDtypeStruct((N_PAD, D), jnp.float32),
        jax.ShapeDtypeStruct((N_PAD,), jnp.int32),
    ),
    mesh=_mesh,
    scratch_types=(
        [
            pltpu.VMEM((NWORK * NC,), jnp.int32),
            pltpu.VMEM((NC,), jnp.int32),
            pltpu.VMEM((NC,), jnp.int32),
            pltpu.VMEM((NC,), jnp.int32),
            pltpu.VMEM((NC,), jnp.int32),
            pltpu.VMEM((NC,), jnp.int32),
            pltpu.VMEM((NC,), jnp.int32),
            pltpu.VMEM((NC, D), jnp.float32),
            pltpu.SemaphoreType.DMA,
        ]
        + [pltpu.SemaphoreType.DMA] * NCH
        + [pltpu.SemaphoreType.DMA] * NCH
    ),
    compiler_params=_cp,
)
def _sc_gather(tbl_hbm, src_hbm, rel_hbm, h_hbm, x_hbm, relout_hbm,
               tb_v, eidx_v, m_v, sg_v, rg_v, sidx_v, rout_v, rows_v,
               sem_t, *sems):
    sems_sr = sems[:NCH]
    sems_h = sems[NCH:]
    wid = lax.axis_index("s") * 2 + lax.axis_index("c")
    nbase = wid * NC
    cps = []
    for j in range(NWORK):
        cps.append(pltpu.async_copy(
            tbl_hbm.at[pl.ds(j * N_PAD + nbase, NC)],
            tb_v.at[pl.ds(j * NC, NC)], sem_t))
    for cp in cps:
        cp.wait()

    cps_sr = []
    for c in range(NCH):
        for t in range(GW // 16):
            k = c * (GW // 16) + t
            m = tb_v[pl.ds(k * 16, 16)]
            for j in range(1, NWORK):
                m = jnp.maximum(m, tb_v[pl.ds(j * NC + k * 16, 16)])
            sl = pl.ds(k * 16, 16)
            m_v[sl] = m
            eidx_v[sl] = jnp.maximum(m, 0)
        slc = pl.ds(c * GW, GW)
        cps_sr.append((
            pltpu.async_copy(src_hbm.at[eidx_v.at[slc]], sg_v.at[slc],
                             sems_sr[c]),
            pltpu.async_copy(rel_hbm.at[eidx_v.at[slc]], rg_v.at[slc],
                             sems_sr[c]),
        ))

    cps_h = []
    for c in range(NCH):
        cps_sr[c][0].wait()
        cps_sr[c][1].wait()
        for t in range(GW // 16):
            k = c * (GW // 16) + t
            sl = pl.ds(k * 16, 16)
            valid = m_v[sl] >= 0
            sidx_v[sl] = jnp.where(valid, sg_v[sl], 0)
            rout_v[sl] = jnp.where(valid, rg_v[sl], R)
        slc = pl.ds(c * GW, GW)
        cps_h.append(pltpu.async_copy(
            h_hbm.at[sidx_v.at[slc]], rows_v.at[slc], sems_h[c]))

    pltpu.sync_copy(rout_v, relout_hbm.at[pl.ds(nbase, NC)])

    for c in range(NCH):
        cps_h[c].wait()
        pltpu.sync_copy(rows_v.at[pl.ds(c * GW, GW)],
                        x_hbm.at[pl.ds(nbase + c * GW, GW)])


def _tc_body(h_ref, x_ref, rel_ref, w_ref, slw_ref, o_ref):
    acc = jnp.dot(h_ref[...], slw_ref[...], preferred_element_type=jnp.float32)
    relv = rel_ref[...]
    x = x_ref[...]
    for r in range(R):
        m = (relv == r).astype(jnp.float32)
        acc = acc + jnp.dot(x * m, w_ref[r], preferred_element_type=jnp.float32)
    o_ref[...] = acc


def _tc_matmul(h, x_pad, rel2, w, slw):
    nb = N_PAD // BLK
    return pl.pallas_call(
        _tc_body,
        grid=(nb,),
        in_specs=[
            pl.BlockSpec((BLK, D), lambda i: (i, 0)),
            pl.BlockSpec((BLK, D), lambda i: (i, 0)),
            pl.BlockSpec((BLK, 1), lambda i: (i, 0)),
            pl.BlockSpec((R, D, D), lambda i: (0, 0, 0)),
            pl.BlockSpec((D, D), lambda i: (0, 0)),
        ],
        out_specs=pl.BlockSpec((BLK, D), lambda i: (i, 0)),
        out_shape=jax.ShapeDtypeStruct((N, D), jnp.float32),
    )(h, x_pad, rel2, w, slw)


def kernel(h, edges, weight, self_loop_weight):
    src = edges[:, 0]
    rel = edges[:, 1]
    dst = edges[:, 2]

    tbl = _sc_lastedge(dst)
    x, relv = _sc_gather(tbl, src, rel, h)
    rel2 = relv.reshape(N_PAD, 1)

    return _tc_matmul(h, x, rel2, weight, self_loop_weight)
